# baseline (speedup 1.0000x reference)
import functools

import jax
import jax.numpy as jnp
from jax import lax
from jax.experimental import pallas as pl
from jax.experimental.pallas import tpu as pltpu
from jax.experimental.pallas import tpu_sc as plsc

E = 8
TILE = 4096
LANES = 16
NUM_WORKERS = 32


def _logits_body(x_ref, w_ref, lt_ref):
    logits = jnp.dot(x_ref[...], w_ref[...], preferred_element_type=jnp.float32)
    lt_ref[...] = logits.T


def _route_chunk(i, lt_v, wv, iv):
    off = i * LANES
    rows = [lt_v[e, pl.ds(off, LANES)] for e in range(E)]
    m1 = rows[0]
    i1 = jnp.zeros((LANES,), jnp.float32)
    m2 = jnp.full((LANES,), -jnp.inf, jnp.float32)
    i2 = jnp.zeros((LANES,), jnp.float32)
    for e in range(1, E):
        v = rows[e]
        ev = jnp.full((LANES,), float(e), jnp.float32)
        b1 = v > m1
        c2 = v > m2
        m2 = jnp.where(b1, m1, jnp.where(c2, v, m2))
        i2 = jnp.where(b1, i1, jnp.where(c2, ev, i2))
        m1 = jnp.where(b1, v, m1)
        i1 = jnp.where(b1, ev, i1)
    z = rows[0] * jnp.float32(0.0)
    for e in range(E):
        z = z + jnp.exp(rows[e] - m1)
    p1 = 1.0 / z
    p2 = jnp.exp(m2 - m1) / z
    w1 = 1.0 / (1.0 + jnp.exp(p2 - p1))
    wv[0, pl.ds(off, LANES)] = w1
    wv[1, pl.ds(off, LANES)] = 1.0 - w1
    iv[0, pl.ds(off, LANES)] = i1.astype(jnp.int32)
    iv[1, pl.ds(off, LANES)] = i2.astype(jnp.int32)


def _make_router_sc(n_tokens):
    chunk = n_tokens // NUM_WORKERS
    mesh = plsc.VectorSubcoreMesh(core_axis_name="c", subcore_axis_name="s")

    @functools.partial(
        pl.kernel,
        mesh=mesh,
        out_type=[
            jax.ShapeDtypeStruct((2, n_tokens), jnp.float32),
            jax.ShapeDtypeStruct((2, n_tokens), jnp.int32),
        ],
        scratch_types=[
            pltpu.VMEM((E, chunk), jnp.float32),
            pltpu.VMEM((2, chunk), jnp.float32),
            pltpu.VMEM((2, chunk), jnp.int32),
        ],
    )
    def route(lt_hbm, wout_hbm, iout_hbm, lt_v, wv, iv):
        wid = lax.axis_index("s") * 2 + lax.axis_index("c")
        base = wid * chunk
        pltpu.sync_copy(lt_hbm.at[:, pl.ds(base, chunk)], lt_v)

        def body(i, carry):
            _route_chunk(i, lt_v, wv, iv)
            return carry

        lax.fori_loop(0, chunk // LANES, body, 0)
        pltpu.sync_copy(wv, wout_hbm.at[:, pl.ds(base, chunk)])
        pltpu.sync_copy(iv, iout_hbm.at[:, pl.ds(base, chunk)])

    return route


def kernel(x, kernel_DE):
    B, T, D = x.shape
    N = B * T
    xf = x.reshape(N, D)
    lt = pl.pallas_call(
        _logits_body,
        grid=(N // TILE,),
        in_specs=[
            pl.BlockSpec((TILE, D), lambda i: (i, 0)),
            pl.BlockSpec((D, E), lambda i: (0, 0)),
        ],
        out_specs=pl.BlockSpec((E, TILE), lambda i: (0, i)),
        out_shape=jax.ShapeDtypeStruct((E, N), jnp.float32),
    )(xf, kernel_DE)
    wout, iout = _make_router_sc(N)(lt)
    return wout.T.reshape(B, T, 2), iout.T.reshape(B, T, 2)
